# baseline (speedup 1.0000x reference)
import functools

import jax
import jax.numpy as jnp
from jax import lax
from jax.experimental import pallas as pl
from jax.experimental.pallas import tpu as pltpu
from jax.experimental.pallas import tpu_sc as plsc

_NUM_EMB = 1024
_EMB_DIM = 64
_COM_COEF = 0.25
_BM = 512


def _tc_body(x_ref, d_ref, idx_ref, loss_ref, perp_ref, hist, acc):
    i = pl.program_id(0)
    nsteps = pl.num_programs(0)
    xb = x_ref[...]
    dm = d_ref[...]
    sim = lax.dot_general(xb, dm, (((1,), (0,)), ((), ())),
                          preferred_element_type=jnp.float32)
    en2 = jnp.sum(dm * dm, axis=0, keepdims=True)
    dist = en2 - 2.0 * sim
    m = jnp.min(dist, axis=1, keepdims=True)
    encf = (dist <= m).astype(jnp.float32)
    iota = lax.broadcasted_iota(jnp.int32, (_NUM_EMB, 6), 0)
    col = lax.broadcasted_iota(jnp.int32, (_NUM_EMB, 6), 1)
    isq = iota * iota
    w6 = jnp.where(col == 0, iota // 256,
         jnp.where(col == 1, iota % 256,
         jnp.where(col == 2, 1,
         jnp.where(col == 3, isq // 65536,
         jnp.where(col == 4, (isq // 256) % 256, isq % 256))))).astype(jnp.float32)
    r6 = lax.dot_general(encf, w6, (((1,), (0,)), ((), ())),
                         preferred_element_type=jnp.float32)
    s = 256.0 * r6[:, 0:1] + r6[:, 1:2]
    cnt = r6[:, 2:3]
    q2 = 65536.0 * r6[:, 3:4] + 256.0 * r6[:, 4:5] + r6[:, 5:6]
    tie1 = 0.5 * (s - jnp.sqrt(jnp.maximum(2.0 * q2 - s * s, 0.0)))
    idxf = jnp.where(cnt > 1.5, tie1, s)
    idx_ref[...] = jnp.clip(idxf.astype(jnp.int32), 0, _NUM_EMB - 1)
    ones_r = jnp.ones((1, _BM), jnp.float32)
    h = lax.dot_general(ones_r, encf, (((1,), (0,)), ((), ())),
                        preferred_element_type=jnp.float32)
    sq = xb * xb
    ones_c = jnp.ones((_EMB_DIM, 1), jnp.float32)
    xn2 = lax.dot_general(sq, ones_c, (((1,), (0,)), ((), ())),
                          preferred_element_type=jnp.float32)
    row_min = m + xn2
    tot = lax.dot_general(ones_r, row_min, (((1,), (0,)), ((), ())),
                          preferred_element_type=jnp.float32)

    @pl.when(i == 0)
    def _():
        acc[0, 0] = 0.0
        hist[...] = jnp.zeros_like(hist)

    acc[0, 0] += tot[0, 0]
    hist[...] += h

    @pl.when(i == nsteps - 1)
    def _():
        n_rows = nsteps * _BM
        loss = (1.0 + _COM_COEF) * acc[0, 0] / (n_rows * _EMB_DIM)
        loss_ref[...] = jnp.full((1, 1), loss, jnp.float32)
        p = hist[...] / n_rows
        perp = jnp.exp(-jnp.sum(p * jnp.log(p + 1e-10)))
        perp_ref[...] = jnp.full((1, 1), perp, jnp.float32)


def _tc_argmin(xf, dictionary):
    n_rows = xf.shape[0]
    grid = n_rows // _BM
    return pl.pallas_call(
        _tc_body,
        grid=(grid,),
        in_specs=[
            pl.BlockSpec((_BM, _EMB_DIM), lambda i: (i, 0)),
            pl.BlockSpec((_EMB_DIM, _NUM_EMB), lambda i: (0, 0)),
        ],
        out_specs=(
            pl.BlockSpec((_BM, 1), lambda i: (i, 0)),
            pl.BlockSpec((1, 1), lambda i: (0, 0)),
            pl.BlockSpec((1, 1), lambda i: (0, 0)),
        ),
        out_shape=(
            jax.ShapeDtypeStruct((n_rows, 1), jnp.int32),
            jax.ShapeDtypeStruct((1, 1), jnp.float32),
            jax.ShapeDtypeStruct((1, 1), jnp.float32),
        ),
        scratch_shapes=[
            pltpu.VMEM((1, _NUM_EMB), jnp.float32),
            pltpu.SMEM((1, 1), jnp.float32),
        ],
    )(xf, dictionary)


def _sc_gather(dict_t, idx3):
    n_rows = idx3.shape[0] * idx3.shape[1] * idx3.shape[2]
    b_per_w = idx3.shape[1] * idx3.shape[2]
    mesh = plsc.VectorSubcoreMesh(core_axis_name="c", subcore_axis_name="s")

    @functools.partial(
        pl.kernel,
        out_type=jax.ShapeDtypeStruct((n_rows, _EMB_DIM), jnp.float32),
        mesh=mesh,
        compiler_params=pltpu.CompilerParams(use_tc_tiling_on_sc=False),
        scratch_types=[
            pltpu.VMEM((8, 128), jnp.int32),
            pltpu.VMEM((b_per_w, _EMB_DIM), jnp.float32),
            pltpu.SemaphoreType.DMA,
        ],
    )
    def k(tab_hbm, idx_hbm, out_hbm, idx_v, rows_v, sem):
        c = lax.axis_index("c")
        s = lax.axis_index("s")
        wid = s * 2 + c
        pltpu.sync_copy(idx_hbm.at[wid], idx_v)
        copies = [
            pltpu.async_copy(tab_hbm.at[idx_v.at[j]],
                             rows_v.at[pl.ds(j * 128, 128)], sem)
            for j in range(8)
        ]
        for cp in copies:
            cp.wait()
        pltpu.sync_copy(rows_v, out_hbm.at[pl.ds(wid * b_per_w, b_per_w)])

    return k(dict_t, idx3)


def kernel(x, dictionary):
    orig_shape = x.shape
    xf = x.reshape(-1, _EMB_DIM)
    idx, loss, perp = _tc_argmin(xf, dictionary)
    q = _sc_gather(dictionary.T, idx.reshape(32, 8, 128))
    return q.reshape(orig_shape), loss[0, 0], perp[0, 0]


# speedup vs baseline: 1.3517x; 1.3517x over previous
import functools

import jax
import jax.numpy as jnp
import numpy as np
from jax import lax
from jax.experimental import pallas as pl
from jax.experimental.pallas import tpu as pltpu
from jax.experimental.pallas import tpu_sc as plsc

_NUM_EMB = 1024
_EMB_DIM = 64
_COM_COEF = 0.25
_BM = 512
_NW = 32
_BPW = 1024


def _tc_body(x_ref, d_ref, idx_ref, loss_ref, perp_ref, hist, acc):
    i = pl.program_id(0)
    nsteps = pl.num_programs(0)
    xb = x_ref[...]
    dm = d_ref[...]
    sim = lax.dot_general(xb, dm, (((1,), (0,)), ((), ())),
                          preferred_element_type=jnp.float32)
    en2 = jnp.sum(dm * dm, axis=0, keepdims=True)
    dist = en2 - 2.0 * sim
    idx = jnp.argmin(dist, axis=1).astype(jnp.int32)
    idx_ref[...] = idx[:, None]
    m = jnp.min(dist, axis=1, keepdims=True)
    onehot = idx[:, None] == lax.broadcasted_iota(jnp.int32, (_BM, _NUM_EMB), 1)
    encf = onehot.astype(jnp.float32)
    ones_r = jnp.ones((1, _BM), jnp.float32)
    h = lax.dot_general(ones_r, encf, (((1,), (0,)), ((), ())),
                        preferred_element_type=jnp.float32)
    sq = xb * xb
    ones_c64 = jnp.ones((_EMB_DIM, 1), jnp.float32)
    xn2 = lax.dot_general(sq, ones_c64, (((1,), (0,)), ((), ())),
                          preferred_element_type=jnp.float32)
    row_min = m + xn2
    tot = lax.dot_general(ones_r, row_min, (((1,), (0,)), ((), ())),
                          preferred_element_type=jnp.float32)

    @pl.when(i == 0)
    def _():
        acc[0, 0] = 0.0
        hist[...] = jnp.zeros_like(hist)

    acc[0, 0] += tot[0, 0]
    hist[...] += h

    @pl.when(i == nsteps - 1)
    def _():
        n_rows = nsteps * _BM
        loss = (1.0 + _COM_COEF) * acc[0, 0] / (n_rows * _EMB_DIM)
        loss_ref[...] = jnp.full((1, 1), loss, jnp.float32)
        p = hist[...] / n_rows
        perp = jnp.exp(-jnp.sum(p * jnp.log(p + 1e-10)))
        perp_ref[...] = jnp.full((1, 1), perp, jnp.float32)


def _tc_argmin(xf, dictionary):
    n_rows = xf.shape[0]
    grid = n_rows // _BM
    return pl.pallas_call(
        _tc_body,
        grid=(grid,),
        in_specs=[
            pl.BlockSpec((_BM, _EMB_DIM), lambda i: (i, 0)),
            pl.BlockSpec((_EMB_DIM, _NUM_EMB), lambda i: (0, 0)),
        ],
        out_specs=(
            pl.BlockSpec((_BM, 1), lambda i: (i, 0)),
            pl.BlockSpec((1, 1), lambda i: (0, 0)),
            pl.BlockSpec((1, 1), lambda i: (0, 0)),
        ),
        out_shape=(
            jax.ShapeDtypeStruct((n_rows, 1), jnp.int32),
            jax.ShapeDtypeStruct((1, 1), jnp.float32),
            jax.ShapeDtypeStruct((1, 1), jnp.float32),
        ),
        scratch_shapes=[
            pltpu.VMEM((1, _NUM_EMB), jnp.float32),
            pltpu.SMEM((1, 1), jnp.float32),
        ],
    )(xf, dictionary)


def _sc_gather(dict_t, idx2):
    mesh = plsc.VectorSubcoreMesh(core_axis_name="c", subcore_axis_name="s")
    n_rows = _NW * _BPW

    @functools.partial(
        pl.kernel,
        out_type=jax.ShapeDtypeStruct((n_rows, _EMB_DIM), jnp.float32),
        mesh=mesh,
        compiler_params=pltpu.CompilerParams(use_tc_tiling_on_sc=False),
        scratch_types=[
            pltpu.VMEM((_BPW,), jnp.int32),
            pltpu.VMEM((_BPW, _EMB_DIM), jnp.float32),
            pltpu.SemaphoreType.DMA,
        ],
    )
    def k(tab_hbm, idx_hbm, out_hbm, idx_v, rows_v, sem):
        c = lax.axis_index("c")
        s = lax.axis_index("s")
        wid = s * 2 + c
        pltpu.sync_copy(idx_hbm.at[wid], idx_v)
        copies = [
            pltpu.async_copy(tab_hbm.at[idx_v.at[pl.ds(j * 128, 128)]],
                             rows_v.at[pl.ds(j * 128, 128)], sem)
            for j in range(8)
        ]
        for cp in copies:
            cp.wait()
        pltpu.sync_copy(rows_v, out_hbm.at[pl.ds(wid * _BPW, _BPW)])

    return k(dict_t, idx2)


def kernel(x, dictionary):
    orig_shape = x.shape
    xf = x.reshape(-1, _EMB_DIM)
    idx, loss, perp = _tc_argmin(xf, dictionary)
    q = _sc_gather(dictionary.T, idx.reshape(_NW, _BPW))
    return q.reshape(orig_shape), loss[0, 0], perp[0, 0]


# speedup vs baseline: 1.5472x; 1.1447x over previous
import functools

import jax
import jax.numpy as jnp
import numpy as np
from jax import lax
from jax.experimental import pallas as pl
from jax.experimental.pallas import tpu as pltpu
from jax.experimental.pallas import tpu_sc as plsc

_NUM_EMB = 1024
_EMB_DIM = 64
_COM_COEF = 0.25
_BM = 1024
_NW = 32
_BPW = 1024


def _tc_body(x_ref, d_ref, idx_ref, loss_ref, perp_ref, hist, acc):
    i = pl.program_id(0)
    nsteps = pl.num_programs(0)
    xb = x_ref[...]
    dm = d_ref[...]
    sim = lax.dot_general(xb, dm, (((1,), (0,)), ((), ())),
                          preferred_element_type=jnp.float32)
    en2 = jnp.sum(dm * dm, axis=0, keepdims=True)
    dist = en2 - 2.0 * sim
    idx = jnp.argmin(dist, axis=1).astype(jnp.int32)
    idx_ref[...] = idx[:, None]
    m = jnp.min(dist, axis=1, keepdims=True)
    onehot = idx[:, None] == lax.broadcasted_iota(jnp.int32, (_BM, _NUM_EMB), 1)
    encf = onehot.astype(jnp.float32)
    ones_r = jnp.ones((1, _BM), jnp.float32)
    h = lax.dot_general(ones_r, encf, (((1,), (0,)), ((), ())),
                        preferred_element_type=jnp.float32)
    sq = xb * xb
    ones_c64 = jnp.ones((_EMB_DIM, 1), jnp.float32)
    xn2 = lax.dot_general(sq, ones_c64, (((1,), (0,)), ((), ())),
                          preferred_element_type=jnp.float32)
    row_min = m + xn2
    tot = lax.dot_general(ones_r, row_min, (((1,), (0,)), ((), ())),
                          preferred_element_type=jnp.float32)

    @pl.when(i == 0)
    def _():
        acc[0, 0] = 0.0
        hist[...] = jnp.zeros_like(hist)

    acc[0, 0] += tot[0, 0]
    hist[...] += h

    @pl.when(i == nsteps - 1)
    def _():
        n_rows = nsteps * _BM
        loss = (1.0 + _COM_COEF) * acc[0, 0] / (n_rows * _EMB_DIM)
        loss_ref[...] = jnp.full((1, 1), loss, jnp.float32)
        p = hist[...] / n_rows
        perp = jnp.exp(-jnp.sum(p * jnp.log(p + 1e-10)))
        perp_ref[...] = jnp.full((1, 1), perp, jnp.float32)


def _tc_argmin(xf, dictionary):
    n_rows = xf.shape[0]
    grid = n_rows // _BM
    return pl.pallas_call(
        _tc_body,
        grid=(grid,),
        in_specs=[
            pl.BlockSpec((_BM, _EMB_DIM), lambda i: (i, 0)),
            pl.BlockSpec((_EMB_DIM, _NUM_EMB), lambda i: (0, 0)),
        ],
        out_specs=(
            pl.BlockSpec((_BM, 1), lambda i: (i, 0)),
            pl.BlockSpec((1, 1), lambda i: (0, 0)),
            pl.BlockSpec((1, 1), lambda i: (0, 0)),
        ),
        out_shape=(
            jax.ShapeDtypeStruct((n_rows, 1), jnp.int32),
            jax.ShapeDtypeStruct((1, 1), jnp.float32),
            jax.ShapeDtypeStruct((1, 1), jnp.float32),
        ),
        scratch_shapes=[
            pltpu.VMEM((1, _NUM_EMB), jnp.float32),
            pltpu.SMEM((1, 1), jnp.float32),
        ],
    )(xf, dictionary)


def _sc_gather(dict_t, idx2):
    mesh = plsc.VectorSubcoreMesh(core_axis_name="c", subcore_axis_name="s")
    n_rows = _NW * _BPW

    @functools.partial(
        pl.kernel,
        out_type=jax.ShapeDtypeStruct((n_rows, _EMB_DIM), jnp.float32),
        mesh=mesh,
        compiler_params=pltpu.CompilerParams(use_tc_tiling_on_sc=False),
        scratch_types=[
            pltpu.VMEM((_BPW,), jnp.int32),
            pltpu.VMEM((_BPW, _EMB_DIM), jnp.float32),
            pltpu.SemaphoreType.DMA,
        ],
    )
    def k(tab_hbm, idx_hbm, out_hbm, idx_v, rows_v, sem):
        c = lax.axis_index("c")
        s = lax.axis_index("s")
        wid = s * 2 + c
        pltpu.sync_copy(idx_hbm.at[wid], idx_v)
        copies = [
            pltpu.async_copy(tab_hbm.at[idx_v.at[pl.ds(j * 128, 128)]],
                             rows_v.at[pl.ds(j * 128, 128)], sem)
            for j in range(8)
        ]
        for cp in copies:
            cp.wait()
        pltpu.sync_copy(rows_v, out_hbm.at[pl.ds(wid * _BPW, _BPW)])

    return k(dict_t, idx2)


def kernel(x, dictionary):
    orig_shape = x.shape
    xf = x.reshape(-1, _EMB_DIM)
    idx, loss, perp = _tc_argmin(xf, dictionary)
    q = _sc_gather(dictionary.T, idx.reshape(_NW, _BPW))
    return q.reshape(orig_shape), loss[0, 0], perp[0, 0]
